# baseline (speedup 1.0000x reference)
import functools

import jax
import jax.numpy as jnp
from jax import lax
from jax.experimental import pallas as pl
from jax.experimental.pallas import tpu as pltpu
from jax.experimental.pallas import tpu_sc as plsc

_N = 32
_H = 32
_K = 5
_LANES = 16
_PAIRS = _H * _N
_PPW = _PAIRS // 32


def _bias_sc_body(sp_hbm, at_hbm, ew_hbm, sw_hbm, out_hbm,
                  sp_idx, e_idx, sw_rows, ew_rows, ct, sem):
    wid = lax.axis_index("s") * 2 + lax.axis_index("c")
    base = wid * _PPW
    pltpu.sync_copy(sp_hbm.at[pl.ds(base, _PPW)], sp_idx)
    for k in range(_K):
        pltpu.sync_copy(at_hbm.at[pl.ds(k * _PAIRS + base, _PPW)], e_idx.at[k])
    copies = [pltpu.async_copy(sw_hbm.at[sp_idx], sw_rows, sem)]
    for k in range(_K):
        copies.append(pltpu.async_copy(ew_hbm.at[e_idx.at[k]], ew_rows.at[k], sem))
    for c in copies:
        c.wait()
    for p in range(_PPW):
        for half in range(2):
            sl = pl.ds(half * _LANES, _LANES)
            r = sw_rows[p, sl]
            for k in range(_K):
                r = r + ew_rows[k, p, sl]
            ct[p, sl] = r
    pltpu.sync_copy(ct, out_hbm.at[pl.ds(base, _PPW)])


def _bias_sc(sp_flat, at_flat, edge_enc_w, spatial_enc_w):
    mesh = plsc.VectorSubcoreMesh(core_axis_name="c", subcore_axis_name="s")
    kern = functools.partial(
        pl.kernel,
        mesh=mesh,
        out_type=jax.ShapeDtypeStruct((_PAIRS, _H), jnp.float32),
        scratch_types=[
            pltpu.VMEM((_PPW,), jnp.int32),
            pltpu.VMEM((_K, _PPW), jnp.int32),
            pltpu.VMEM((_PPW, 128), jnp.float32),
            pltpu.VMEM((_K, _PPW, 128), jnp.float32),
            pltpu.VMEM((_PPW, _H), jnp.float32),
            pltpu.SemaphoreType.DMA,
        ],
    )(_bias_sc_body)
    return kern(sp_flat, at_flat, edge_enc_w, spatial_enc_w)


def _assemble_body(ct_ref, at_ref, v_ref, c3_ref):
    r3 = ct_ref[...].reshape(_H, _N, _N)
    c3 = jnp.transpose(r3, (2, 0, 1))
    a2 = at_ref[...].sum(0).astype(jnp.float32)
    c3 = c3 + a2[None, :, :]
    ii = lax.broadcasted_iota(jnp.int32, (_N, _H, _N), 0)
    jj = lax.broadcasted_iota(jnp.int32, (_N, _H, _N), 2)
    vb = jnp.broadcast_to(v_ref[...], (_H, _N))[None, :, :]
    c3_ref[...] = c3 + jnp.where((ii == 0) | (jj == 0), vb, 0.0)


def _assemble_tc(ct, atnet, v_col):
    return pl.pallas_call(
        _assemble_body,
        out_shape=jax.ShapeDtypeStruct((_N, _H, _N), jnp.float32),
    )(ct, atnet, v_col)


def _add_body(ab_ref, c_ref, o_ref):
    o_ref[...] = ab_ref[...][:, :, None, :] + c_ref[...][None]


def _add_flat_body(ab_ref, c_ref, o_ref):
    ab = ab_ref[...]
    tiled = jnp.concatenate([ab] * _H, axis=-1)
    o_ref[...] = tiled + c_ref[...][None]


_GB = 64
_NBUF = 2
_NSPLIT = 8
_SPG = _GB // _NSPLIT


def _add_manual_body(ab_ref, c_ref, o_hbm, buf, sem):
    g = pl.program_id(0)
    nsteps = pl.num_programs(0)
    slot = lax.rem(g, _NBUF)

    @pl.when(g >= _NBUF)
    def _wait_prev():
        for s in range(_NSPLIT):
            pltpu.make_async_copy(
                buf.at[slot, pl.ds(s * _SPG, _SPG)],
                o_hbm.at[pl.ds(0, _SPG)], sem.at[slot, s]
            ).wait()

    ab = ab_ref[...]
    buf[slot] = jnp.concatenate([ab] * _H, axis=-1) + c_ref[...][None]
    for s in range(_NSPLIT):
        pltpu.make_async_copy(
            buf.at[slot, pl.ds(s * _SPG, _SPG)],
            o_hbm.at[pl.ds(g * _GB + s * _SPG, _SPG)], sem.at[slot, s]
        ).start()

    @pl.when(g == nsteps - 1)
    def _drain():
        for b in range(_NBUF):
            for s in range(_NSPLIT):
                pltpu.make_async_copy(
                    buf.at[b, pl.ds(s * _SPG, _SPG)],
                    o_hbm.at[pl.ds(0, _SPG)], sem.at[b, s]
                ).wait()


_ROW = _H * _N
_GROW = _N * _ROW


def _make_add_sc_body(gpw):
    def _add_sc_body(ab_hbm, c_hbm, out_hbm, c_v, ab_v, ob_v, absem, osem):
        wid = lax.axis_index("s") * 2 + lax.axis_index("c")
        gbase = wid * gpw
        pltpu.sync_copy(c_hbm, c_v)
        pltpu.make_async_copy(ab_hbm.at[gbase], ab_v.at[pl.ds(0, _ROW)],
                              absem.at[0]).start()

        def graph_body(gl, carry):
            slot = lax.rem(gl, 2)
            abase = slot * _ROW
            obase = slot * _GROW
            pltpu.make_async_copy(ab_hbm.at[gbase], ab_v.at[pl.ds(abase, _ROW)],
                                  absem.at[slot]).wait()

            @pl.when(gl < gpw - 1)
            def _prefetch():
                nslot = 1 - slot
                pltpu.make_async_copy(
                    ab_hbm.at[gbase + gl + 1],
                    ab_v.at[pl.ds(nslot * _ROW, _ROW)], absem.at[nslot]).start()

            @pl.when(gl >= 2)
            def _wait_out():
                pltpu.make_async_copy(
                    ob_v.at[pl.ds(obase, _GROW)], out_hbm.at[gbase],
                    osem.at[slot]).wait()

            def row_body(i, c2):
                lo = ab_v[pl.ds(abase + i * _N, _LANES)]
                hi = ab_v[pl.ds(abase + i * _N + _LANES, _LANES)]
                for c in range(_ROW // _LANES):
                    off = i * _ROW + c * _LANES
                    src = lo if c % 2 == 0 else hi
                    ob_v[pl.ds(obase + off, _LANES)] = c_v[pl.ds(off, _LANES)] + src
                return c2

            lax.fori_loop(0, _N, row_body, 0)
            pltpu.make_async_copy(ob_v.at[pl.ds(obase, _GROW)],
                                  out_hbm.at[gbase + gl], osem.at[slot]).start()
            return carry

        lax.fori_loop(0, gpw, graph_body, 0)
        for s in range(2):
            pltpu.make_async_copy(ob_v.at[pl.ds(s * _GROW, _GROW)],
                                  out_hbm.at[gbase], osem.at[s]).wait()

    return _add_sc_body


def _bias_add_sc(ab2, c_flat):
    gpw = ab2.shape[0] // 32
    mesh = plsc.VectorSubcoreMesh(core_axis_name="c", subcore_axis_name="s")
    kern = functools.partial(
        pl.kernel,
        mesh=mesh,
        out_type=jax.ShapeDtypeStruct((ab2.shape[0], _GROW), jnp.float32),
        scratch_types=[
            pltpu.VMEM((_GROW,), jnp.float32),
            pltpu.VMEM((2 * _ROW,), jnp.float32),
            pltpu.VMEM((2 * _GROW,), jnp.float32),
            pltpu.SemaphoreType.DMA((2,)),
            pltpu.SemaphoreType.DMA((2,)),
        ],
    )(_make_add_sc_body(gpw))
    return kern(ab2, c_flat)


def _bias_add_manual_tc(ab, c2):
    ng = ab.shape[0]
    out = pl.pallas_call(
        _add_manual_body,
        grid=(ng // _GB,),
        in_specs=[
            pl.BlockSpec((_GB, _N, _N), lambda g: (g, 0, 0)),
            pl.BlockSpec((_N, _H * _N), lambda g: (0, 0)),
        ],
        out_specs=pl.BlockSpec(memory_space=pl.ANY),
        out_shape=jax.ShapeDtypeStruct((ng, _N, _H * _N), jnp.float32),
        scratch_shapes=[
            pltpu.VMEM((_NBUF, _GB, _N, _H * _N), jnp.float32),
            pltpu.SemaphoreType.DMA((_NBUF, _NSPLIT)),
        ],
        compiler_params=pltpu.CompilerParams(
            dimension_semantics=("arbitrary",),
        ),
    )(ab, c2)
    return out.reshape(ng, _N, _H, _N)


def _bias_add_flat_tc(ab, c2):
    ng = ab.shape[0]
    gb = 64
    out = pl.pallas_call(
        _add_flat_body,
        grid=(ng // gb,),
        in_specs=[
            pl.BlockSpec((gb, _N, _N), lambda g: (g, 0, 0)),
            pl.BlockSpec((_N, _H * _N), lambda g: (0, 0)),
        ],
        out_specs=pl.BlockSpec((gb, _N, _H * _N), lambda g: (g, 0, 0)),
        out_shape=jax.ShapeDtypeStruct((ng, _N, _H * _N), jnp.float32),
        compiler_params=pltpu.CompilerParams(
            dimension_semantics=("parallel",),
        ),
    )(ab, c2)
    return out.reshape(ng, _N, _H, _N)


def _bias_add_tc(ab, c3):
    ng = ab.shape[0]
    gb = 8
    return pl.pallas_call(
        _add_body,
        grid=(ng // gb,),
        in_specs=[
            pl.BlockSpec((gb, _N, _N), lambda g: (g, 0, 0)),
            pl.BlockSpec((_N, _H, _N), lambda g: (0, 0, 0)),
        ],
        out_specs=pl.BlockSpec((gb, _N, _H, _N), lambda g: (g, 0, 0, 0)),
        out_shape=jax.ShapeDtypeStruct((ng, _N, _H, _N), jnp.float32),
        compiler_params=pltpu.CompilerParams(
            dimension_semantics=("parallel",),
        ),
    )(ab, c3)


def kernel(ab, sp, nf, ei, atnet, edge_enc_w, spatial_enc_w, virt_dist_w):
    del nf, ei
    sp_flat = sp.reshape(-1).astype(jnp.int32)
    at_t = jnp.transpose(atnet, (2, 0, 1)).astype(jnp.int32)
    at_flat = at_t.reshape(-1)
    ew_p = jnp.pad(edge_enc_w.astype(jnp.float32), ((0, 0), (0, 128 - _H)))
    sw_p = jnp.pad(spatial_enc_w.astype(jnp.float32), ((0, 0), (0, 128 - _H)))
    ct = _bias_sc(sp_flat, at_flat, ew_p, sw_p)
    v_col = virt_dist_w.reshape(_H, 1).astype(jnp.float32)
    c3 = _assemble_tc(ct, at_t, v_col)
    ng = ab.shape[0]
    n_tc = 704
    out_tc = _bias_add_flat_tc(ab[:n_tc], c3.reshape(_N, _H * _N))
    out_sc = _bias_add_sc(ab.reshape(ng, _N * _N)[n_tc:], c3.reshape(-1))
    out = jnp.concatenate([out_tc.reshape(n_tc, _GROW), out_sc], axis=0)
    return out.reshape(ng, _N, _H, _N)


# speedup vs baseline: 1.4356x; 1.4356x over previous
import functools

import jax
import jax.numpy as jnp
from jax import lax
from jax.experimental import pallas as pl
from jax.experimental.pallas import tpu as pltpu
from jax.experimental.pallas import tpu_sc as plsc

_N = 32
_H = 32
_K = 5
_LANES = 16
_PAIRS = _H * _N
_PPW = _PAIRS // 32


def _bias_sc_body(sp_hbm, at_hbm, ew_hbm, sw_hbm, out_hbm,
                  sp_idx, e_idx, sw_rows, ew_rows, ct, sem):
    wid = lax.axis_index("s") * 2 + lax.axis_index("c")
    base = wid * _PPW
    pltpu.sync_copy(sp_hbm.at[pl.ds(base, _PPW)], sp_idx)
    for k in range(_K):
        pltpu.sync_copy(at_hbm.at[pl.ds(k * _PAIRS + base, _PPW)], e_idx.at[k])
    copies = [pltpu.async_copy(sw_hbm.at[sp_idx], sw_rows, sem)]
    for k in range(_K):
        copies.append(pltpu.async_copy(ew_hbm.at[e_idx.at[k]], ew_rows.at[k], sem))
    for c in copies:
        c.wait()
    for p in range(_PPW):
        for half in range(2):
            sl = pl.ds(half * _LANES, _LANES)
            r = sw_rows[p, sl]
            for k in range(_K):
                r = r + ew_rows[k, p, sl]
            ct[p, sl] = r
    pltpu.sync_copy(ct, out_hbm.at[pl.ds(base, _PPW)])


def _bias_sc(sp_flat, at_flat, ew_padded, sw_padded):
    mesh = plsc.VectorSubcoreMesh(core_axis_name="c", subcore_axis_name="s")
    kern = functools.partial(
        pl.kernel,
        mesh=mesh,
        out_type=jax.ShapeDtypeStruct((_PAIRS, _H), jnp.float32),
        scratch_types=[
            pltpu.VMEM((_PPW,), jnp.int32),
            pltpu.VMEM((_K, _PPW), jnp.int32),
            pltpu.VMEM((_PPW, 128), jnp.float32),
            pltpu.VMEM((_K, _PPW, 128), jnp.float32),
            pltpu.VMEM((_PPW, _H), jnp.float32),
            pltpu.SemaphoreType.DMA,
        ],
    )(_bias_sc_body)
    return kern(sp_flat, at_flat, ew_padded, sw_padded)


def _assemble_body(ct_ref, at_ref, v_ref, c3_ref):
    r3 = ct_ref[...].reshape(_H, _N, _N)
    c3 = jnp.transpose(r3, (2, 0, 1))
    a2 = at_ref[...].sum(0).astype(jnp.float32)
    c3 = c3 + a2[None, :, :]
    ii = lax.broadcasted_iota(jnp.int32, (_N, _H, _N), 0)
    jj = lax.broadcasted_iota(jnp.int32, (_N, _H, _N), 2)
    vb = jnp.broadcast_to(v_ref[...], (_H, _N))[None, :, :]
    c3_ref[...] = c3 + jnp.where((ii == 0) | (jj == 0), vb, 0.0)


def _assemble_tc(ct, at_t, v_col):
    return pl.pallas_call(
        _assemble_body,
        out_shape=jax.ShapeDtypeStruct((_N, _H, _N), jnp.float32),
    )(ct, at_t, v_col)


def _add_flat_body(ab_ref, c_ref, o_ref):
    ab = ab_ref[...]
    tiled = jnp.concatenate([ab] * _H, axis=-1)
    o_ref[...] = tiled + c_ref[...][None]


def _bias_add_flat_tc(ab, c2):
    ng = ab.shape[0]
    gb = 128
    out = pl.pallas_call(
        _add_flat_body,
        grid=(ng // gb,),
        in_specs=[
            pl.BlockSpec((gb, _N, _N), lambda g: (g, 0, 0)),
            pl.BlockSpec((_N, _H * _N), lambda g: (0, 0)),
        ],
        out_specs=pl.BlockSpec((gb, _N, _H * _N), lambda g: (g, 0, 0)),
        out_shape=jax.ShapeDtypeStruct((ng, _N, _H * _N), jnp.float32),
        compiler_params=pltpu.CompilerParams(
            dimension_semantics=("parallel",),
        ),
    )(ab, c2)
    return out.reshape(ng, _N, _H, _N)


def kernel(ab, sp, nf, ei, atnet, edge_enc_w, spatial_enc_w, virt_dist_w):
    del nf, ei
    sp_flat = sp.reshape(-1).astype(jnp.int32)
    at_t = jnp.transpose(atnet, (2, 0, 1)).astype(jnp.int32)
    at_flat = at_t.reshape(-1)
    ew_p = jnp.pad(edge_enc_w.astype(jnp.float32), ((0, 0), (0, 128 - _H)))
    sw_p = jnp.pad(spatial_enc_w.astype(jnp.float32), ((0, 0), (0, 128 - _H)))
    ct = _bias_sc(sp_flat, at_flat, ew_p, sw_p)
    v_col = virt_dist_w.reshape(_H, 1).astype(jnp.float32)
    c3 = _assemble_tc(ct, at_t, v_col)
    return _bias_add_flat_tc(ab, c3.reshape(_N, _H * _N))
